# baseline (speedup 1.0000x reference)
import functools

import jax
import jax.numpy as jnp
from jax import lax
from jax.experimental import pallas as pl
from jax.experimental.pallas import tpu as pltpu
from jax.experimental.pallas import tpu_sc as plsc

NC = 2
NS = 16
L = 16

CLS = 20
L_COORD = 5.0
L_NOOBJ = 0.5


@functools.lru_cache(maxsize=None)
def _build_sc_loss(batch: int, s1: int, s2: int):
    assert batch % (16 * L) == 0 and s1 == 7 and s2 == 7
    rows = 4
    mesh = plsc.VectorSubcoreMesh(core_axis_name="c", subcore_axis_name="s",
                                  num_cores=NC, num_subcores=NS)

    def body(cls_hbm, resp_hbm, pb_hbm, lb_hbm, out_hbm,
             pc_v, lc_v, pr_v, lr_v, pb_v, lb_v, part_v, sem1, sem2):
        cid = lax.axis_index("c")
        sid = lax.axis_index("s")
        wid = sid * NC + cid
        bg = lax.rem(wid, 16)
        half = lax.div(wid, 16)
        b0 = bg * L
        bt = lax.div(bg, 8)
        lo = lax.rem(bg, 8) * L
        r0 = half * 3
        lanes = lax.iota(jnp.int32, L)
        zero = jnp.zeros((L,), jnp.float32)

        c_lr = pltpu.async_copy(
            resp_hbm.at[1, pl.ds(r0, rows), :, bt, :, pl.ds(lo, L)], lr_v, sem1)
        c_pr = pltpu.async_copy(
            resp_hbm.at[0, pl.ds(r0, rows), :, bt, :, pl.ds(lo, L)], pr_v, sem1)
        c_pb = pltpu.async_copy(
            pb_hbm.at[pl.ds(r0, rows), :, bt, :, pl.ds(lo, L)], pb_v, sem1)
        c_lb = pltpu.async_copy(
            lb_hbm.at[pl.ds(r0, rows), :, bt, :, pl.ds(lo, L)], lb_v, sem1)
        c_pc = pltpu.async_copy(
            cls_hbm.at[0, pl.ds(r0, rows), :, :, pl.ds(b0, L)], pc_v, sem2)
        c_lc = pltpu.async_copy(
            cls_hbm.at[1, pl.ds(r0, rows), :, :, pl.ds(b0, L)], lc_v, sem2)
        c_lr.wait()
        c_pr.wait()
        c_pb.wait()
        c_lb.wait()

        is_h1 = half > 0
        ncols = rows * s2

        def col_mask(j, r, c2):
            on_edge = jnp.where(is_h1, r == 0, r == rows - 1)
            own = jnp.where(is_h1, c2 >= 4, c2 < 4)
            w = jnp.where(on_edge & ~own, 0.0, 1.0)
            lr0 = lr_v[r, c2, 0, :]
            return jnp.where(lr0 > 0.0, w, 0.0), w

        def column_iou(j, accs):
            noobj_a, loc_a, pobj_a, iou_a = accs
            r = lax.div(j, s2)
            c2 = lax.rem(j, s2)
            m, w = col_mask(j, r, c2)
            obj = m > 0.0
            lr0 = lr_v[r, c2, 0, :]
            lr1 = lr_v[r, c2, 1, :]
            pr0 = pr_v[r, c2, 0, :]
            pr1 = pr_v[r, c2, 1, :]

            def corners(ref, k0):
                x = ref[r, c2, k0, :]
                y = ref[r, c2, k0 + 1, :]
                bw = ref[r, c2, k0 + 2, :]
                bh = ref[r, c2, k0 + 3, :]
                hw = 0.5 * (bw * bw)
                hh = 0.5 * (bh * bh)
                return x - hw, y - hh, x + hw, y + hh

            def iou_for(k0):
                tx1, ty1, tx2, ty2 = corners(lb_v, k0)
                px1, py1, px2, py2 = corners(pb_v, k0)
                ltx = jnp.maximum(tx1, px1)
                lty = jnp.maximum(ty1, py1)
                rbx = jnp.minimum(tx2, px2)
                rby = jnp.minimum(ty2, py2)
                wx = jnp.maximum(rbx - ltx, 0.0)
                wy = jnp.maximum(rby - lty, 0.0)
                inter = wx * wy
                a1 = (tx2 - tx1) * (ty2 - ty1)
                a2 = (px2 - px1) * (py2 - py1)
                return jnp.where(obj, inter / (a1 + a2 - inter), 0.0)

            iou0 = iou_for(0)
            iou1 = iou_for(4)
            maxiou = jnp.maximum(iou0, iou1)
            sel1 = iou1 > iou0

            def sel(a0, a1):
                return jnp.where(sel1, a1, a0)

            loc_t = zero
            for k in range(4):
                dk = (sel(pb_v[r, c2, k, :], pb_v[r, c2, 4 + k, :])
                      - sel(lb_v[r, c2, k, :], lb_v[r, c2, 4 + k, :]))
                loc_t = loc_t + dk * dk
            loc_a = loc_a + m * loc_t
            dpo = sel(pr0, pr1) - maxiou
            dio = maxiou - sel(lr0, lr1)
            pobj_a = pobj_a + m * (dpo * dpo)
            iou_a = iou_a + m * (dio * dio)
            nm = w - m
            d0 = pr0 - lr0
            d1 = pr1 - lr1
            noobj_a = noobj_a + nm * (d0 * d0 + d1 * d1)
            return noobj_a, loc_a, pobj_a, iou_a

        accs = lax.fori_loop(0, ncols, column_iou, (zero, zero, zero, zero))
        noobj_a, loc_a, pobj_a, iou_a = accs

        c_pc.wait()
        c_lc.wait()

        def column_cls(j, cls_a):
            r = lax.div(j, s2)
            c2 = lax.rem(j, s2)
            m, _ = col_mask(j, r, c2)
            csum = zero
            for c in range(CLS):
                d = pc_v[r, c, c2, :] - lc_v[r, c, c2, :]
                csum = csum + d * d
            return cls_a + m * csum

        cls_a = lax.fori_loop(0, ncols, column_cls, zero)

        inv = 1.0 / batch
        s_off = (L_COORD * inv) * jnp.sum(loc_a)
        s_cls = inv * jnp.sum(cls_a)
        s_pobj = inv * jnp.sum(pobj_a)
        s_nobj = (L_NOOBJ * inv) * jnp.sum(noobj_a)
        s_iou = inv * jnp.sum(iou_a)
        res = (jnp.where(lanes == 0, s_off, 0.0)
               + jnp.where(lanes == 1, s_cls, 0.0)
               + jnp.where(lanes == 2, s_pobj, 0.0)
               + jnp.where(lanes == 3, s_nobj, 0.0)
               + jnp.where(lanes == 4, s_iou, 0.0))
        for i in range(8):
            for c in range(0, 128, L):
                part_v[i, pl.ds(c, L)] = res if (i == 0 and c == 0) else zero
        pltpu.sync_copy(part_v, out_hbm.at[wid])

    return pl.kernel(
        body,
        out_type=jax.ShapeDtypeStruct((NC * NS, 8, 128), jnp.float32),
        mesh=mesh,
        scratch_types=[
            pltpu.VMEM((rows, CLS, s2, L), jnp.float32),
            pltpu.VMEM((rows, CLS, s2, L), jnp.float32),
            pltpu.VMEM((rows, s2, 2, L), jnp.float32),
            pltpu.VMEM((rows, s2, 2, L), jnp.float32),
            pltpu.VMEM((rows, s2, 8, L), jnp.float32),
            pltpu.VMEM((rows, s2, 8, L), jnp.float32),
            pltpu.VMEM((8, 128), jnp.float32),
            pltpu.SemaphoreType.DMA,
            pltpu.SemaphoreType.DMA,
        ],
        compiler_params=pltpu.CompilerParams(needs_layout_passes=False,
                                             use_tc_tiling_on_sc=False),
    )


def _resp_view(x):
    k = x.shape[-1]
    return jnp.transpose(x.reshape(2, 128, 7, 7, k), (2, 3, 0, 4, 1))


def kernel(pred_cls, pred_response, pred_bboxes, label_cls, label_response,
           label_bboxes):
    batch, s1, s2 = pred_cls.shape[0], pred_cls.shape[1], pred_cls.shape[2]
    fn = _build_sc_loss(batch, s1, s2)
    cls_st = jnp.stack([jnp.transpose(pred_cls, (1, 3, 2, 0)),
                        jnp.transpose(label_cls, (1, 3, 2, 0))])
    resp_st = jnp.stack([_resp_view(pred_response),
                         _resp_view(label_response)])
    out = fn(cls_st, resp_st, _resp_view(pred_bboxes),
             _resp_view(label_bboxes))
    s = jnp.sum(out, axis=(0, 1))
    return {"offset": s[0], "cls": s[1], "pObj": s[2], "nObj": s[3],
            "iou": s[4]}


# speedup vs baseline: 1.0086x; 1.0086x over previous
import functools

import jax
import jax.numpy as jnp
from jax import lax
from jax.experimental import pallas as pl
from jax.experimental.pallas import tpu as pltpu
from jax.experimental.pallas import tpu_sc as plsc

NC = 2
NS = 16
L = 16

CLS = 20
L_COORD = 5.0
L_NOOBJ = 0.5


@functools.lru_cache(maxsize=None)
def _build_sc_loss(batch: int, s1: int, s2: int):
    assert batch % (16 * L) == 0 and s1 == 7 and s2 == 7
    rows = 4
    mesh = plsc.VectorSubcoreMesh(core_axis_name="c", subcore_axis_name="s",
                                  num_cores=NC, num_subcores=NS)

    def body(cls_hbm, resp_hbm, pb_hbm, lb_hbm, out_hbm,
             pc_v, lc_v, pr_v, lr_v, pb_v, lb_v, part_v, sem1, sem2):
        cid = lax.axis_index("c")
        sid = lax.axis_index("s")
        wid = sid * NC + cid
        bg = lax.rem(wid, 16)
        half = lax.div(wid, 16)
        b0 = bg * L
        bt = lax.div(bg, 8)
        lo = lax.rem(bg, 8) * L
        r0 = half * 3
        lanes = lax.iota(jnp.int32, L)
        zero = jnp.zeros((L,), jnp.float32)

        c_lr = pltpu.async_copy(
            resp_hbm.at[1, pl.ds(r0, rows), :, bt, :, pl.ds(lo, L)], lr_v, sem1)
        c_pr = pltpu.async_copy(
            resp_hbm.at[0, pl.ds(r0, rows), :, bt, :, pl.ds(lo, L)], pr_v, sem1)
        c_pb = pltpu.async_copy(
            pb_hbm.at[pl.ds(r0, rows), :, bt, :, pl.ds(lo, L)], pb_v, sem1)
        c_lb = pltpu.async_copy(
            lb_hbm.at[pl.ds(r0, rows), :, bt, :, pl.ds(lo, L)], lb_v, sem1)
        c_pc = pltpu.async_copy(
            cls_hbm.at[0, pl.ds(r0, rows), :, :, pl.ds(b0, L)], pc_v, sem2)
        c_lc = pltpu.async_copy(
            cls_hbm.at[1, pl.ds(r0, rows), :, :, pl.ds(b0, L)], lc_v, sem2)
        c_lr.wait()
        c_pr.wait()
        c_pb.wait()
        c_lb.wait()

        is_h1 = half > 0
        ncols = rows * s2

        def col_mask(j, r, c2):
            on_edge = jnp.where(is_h1, r == 0, r == rows - 1)
            own = jnp.where(is_h1, c2 >= 4, c2 < 4)
            w = jnp.where(on_edge & ~own, 0.0, 1.0)
            lr0 = lr_v[r, c2, 0, :]
            return jnp.where(lr0 > 0.0, w, 0.0), w

        def column_iou(j, accs):
            noobj_a, loc_a, pobj_a, iou_a = accs
            r = lax.div(j, s2)
            c2 = lax.rem(j, s2)
            m, w = col_mask(j, r, c2)
            obj = m > 0.0
            lr0 = lr_v[r, c2, 0, :]
            lr1 = lr_v[r, c2, 1, :]
            pr0 = pr_v[r, c2, 0, :]
            pr1 = pr_v[r, c2, 1, :]

            def corners(ref, k0):
                x = ref[r, c2, k0, :]
                y = ref[r, c2, k0 + 1, :]
                bw = ref[r, c2, k0 + 2, :]
                bh = ref[r, c2, k0 + 3, :]
                hw = 0.5 * (bw * bw)
                hh = 0.5 * (bh * bh)
                return x - hw, y - hh, x + hw, y + hh

            def iou_for(k0):
                tx1, ty1, tx2, ty2 = corners(lb_v, k0)
                px1, py1, px2, py2 = corners(pb_v, k0)
                ltx = jnp.maximum(tx1, px1)
                lty = jnp.maximum(ty1, py1)
                rbx = jnp.minimum(tx2, px2)
                rby = jnp.minimum(ty2, py2)
                wx = jnp.maximum(rbx - ltx, 0.0)
                wy = jnp.maximum(rby - lty, 0.0)
                inter = wx * wy
                a1 = (tx2 - tx1) * (ty2 - ty1)
                a2 = (px2 - px1) * (py2 - py1)
                return jnp.where(obj, inter / (a1 + a2 - inter), 0.0)

            iou0 = iou_for(0)
            iou1 = iou_for(4)
            maxiou = jnp.maximum(iou0, iou1)
            sel1 = iou1 > iou0

            def sel(a0, a1):
                return jnp.where(sel1, a1, a0)

            loc_t = zero
            for k in range(4):
                dk = (sel(pb_v[r, c2, k, :], pb_v[r, c2, 4 + k, :])
                      - sel(lb_v[r, c2, k, :], lb_v[r, c2, 4 + k, :]))
                loc_t = loc_t + dk * dk
            loc_a = loc_a + m * loc_t
            dpo = sel(pr0, pr1) - maxiou
            dio = maxiou - sel(lr0, lr1)
            pobj_a = pobj_a + m * (dpo * dpo)
            iou_a = iou_a + m * (dio * dio)
            nm = w - m
            d0 = pr0 - lr0
            d1 = pr1 - lr1
            noobj_a = noobj_a + nm * (d0 * d0 + d1 * d1)
            return noobj_a, loc_a, pobj_a, iou_a

        accs = lax.fori_loop(0, ncols, column_iou, (zero, zero, zero, zero))
        noobj_a, loc_a, pobj_a, iou_a = accs

        c_pc.wait()
        c_lc.wait()

        def column_cls(j, cls_a):
            r = lax.div(j, s2)
            c2 = lax.rem(j, s2)
            m, _ = col_mask(j, r, c2)
            csum = zero
            for c in range(CLS):
                d = pc_v[r, c, c2, :] - lc_v[r, c, c2, :]
                csum = csum + d * d
            return cls_a + m * csum

        cls_a = lax.fori_loop(0, ncols, column_cls, zero)

        inv = 1.0 / batch
        s_off = (L_COORD * inv) * jnp.sum(loc_a)
        s_cls = inv * jnp.sum(cls_a)
        s_pobj = inv * jnp.sum(pobj_a)
        s_nobj = (L_NOOBJ * inv) * jnp.sum(noobj_a)
        s_iou = inv * jnp.sum(iou_a)
        res = (jnp.where(lanes == 0, s_off, 0.0)
               + jnp.where(lanes == 1, s_cls, 0.0)
               + jnp.where(lanes == 2, s_pobj, 0.0)
               + jnp.where(lanes == 3, s_nobj, 0.0)
               + jnp.where(lanes == 4, s_iou, 0.0))
        part_v[...] = res
        pltpu.sync_copy(part_v, out_hbm.at[wid])

    return pl.kernel(
        body,
        out_type=jax.ShapeDtypeStruct((NC * NS, L), jnp.float32),
        mesh=mesh,
        scratch_types=[
            pltpu.VMEM((rows, CLS, s2, L), jnp.float32),
            pltpu.VMEM((rows, CLS, s2, L), jnp.float32),
            pltpu.VMEM((rows, s2, 2, L), jnp.float32),
            pltpu.VMEM((rows, s2, 2, L), jnp.float32),
            pltpu.VMEM((rows, s2, 8, L), jnp.float32),
            pltpu.VMEM((rows, s2, 8, L), jnp.float32),
            pltpu.VMEM((L,), jnp.float32),
            pltpu.SemaphoreType.DMA,
            pltpu.SemaphoreType.DMA,
        ],
        compiler_params=pltpu.CompilerParams(needs_layout_passes=False,
                                             use_tc_tiling_on_sc=False),
    )


def _resp_view(x):
    k = x.shape[-1]
    return jnp.transpose(x.reshape(2, 128, 7, 7, k), (2, 3, 0, 4, 1))


def kernel(pred_cls, pred_response, pred_bboxes, label_cls, label_response,
           label_bboxes):
    batch, s1, s2 = pred_cls.shape[0], pred_cls.shape[1], pred_cls.shape[2]
    fn = _build_sc_loss(batch, s1, s2)
    cls_st = jnp.stack([jnp.transpose(pred_cls, (1, 3, 2, 0)),
                        jnp.transpose(label_cls, (1, 3, 2, 0))])
    resp_st = jnp.stack([_resp_view(pred_response),
                         _resp_view(label_response)])
    out = fn(cls_st, resp_st, _resp_view(pred_bboxes),
             _resp_view(label_bboxes))
    s = jnp.sum(out, axis=0)
    return {"offset": s[0], "cls": s[1], "pObj": s[2], "nObj": s[3],
            "iou": s[4]}


# speedup vs baseline: 1.0093x; 1.0006x over previous
import functools

import jax
import jax.numpy as jnp
from jax import lax
from jax.experimental import pallas as pl
from jax.experimental.pallas import tpu as pltpu
from jax.experimental.pallas import tpu_sc as plsc

NC = 2
NS = 16
L = 16

CLS = 20
L_COORD = 5.0
L_NOOBJ = 0.5


@functools.lru_cache(maxsize=None)
def _build_sc_loss(batch: int, s1: int, s2: int):
    assert batch % (16 * L) == 0 and s1 == 7 and s2 == 7
    rows = 4
    mesh = plsc.VectorSubcoreMesh(core_axis_name="c", subcore_axis_name="s",
                                  num_cores=NC, num_subcores=NS)

    def body(cls_hbm, resp_hbm, pb_hbm, lb_hbm, out_hbm,
             pc_v, lc_v, pr_v, lr_v, pb_v, lb_v, part_v, sem1, sem2):
        cid = lax.axis_index("c")
        sid = lax.axis_index("s")
        wid = sid * NC + cid
        bg = lax.rem(wid, 16)
        half = lax.div(wid, 16)
        b0 = bg * L
        bt = lax.div(bg, 8)
        lo = lax.rem(bg, 8) * L
        r0 = half * 3
        lanes = lax.iota(jnp.int32, L)
        zero = jnp.zeros((L,), jnp.float32)

        c_lr = pltpu.async_copy(
            resp_hbm.at[1, pl.ds(r0, rows), :, bt, :, pl.ds(lo, L)], lr_v, sem1)
        c_pr = pltpu.async_copy(
            resp_hbm.at[0, pl.ds(r0, rows), :, bt, :, pl.ds(lo, L)], pr_v, sem1)
        c_pb = pltpu.async_copy(
            pb_hbm.at[pl.ds(r0, rows), :, bt, :, pl.ds(lo, L)], pb_v, sem1)
        c_lb = pltpu.async_copy(
            lb_hbm.at[pl.ds(r0, rows), :, bt, :, pl.ds(lo, L)], lb_v, sem1)
        c_pc = pltpu.async_copy(
            cls_hbm.at[0, pl.ds(r0, rows), :, :, pl.ds(b0, L)], pc_v, sem2)
        c_lc = pltpu.async_copy(
            cls_hbm.at[1, pl.ds(r0, rows), :, :, pl.ds(b0, L)], lc_v, sem2)
        c_lr.wait()
        c_pr.wait()
        c_pb.wait()
        c_lb.wait()

        is_h1 = half > 0
        ncols = rows * s2

        def col_mask(r, c2):
            on_edge = jnp.where(is_h1, r == 0, r == rows - 1)
            own = jnp.where(is_h1, c2 >= 4, c2 < 4)
            w = jnp.where(on_edge & ~own, 0.0, 1.0)
            lr0 = lr_v[r, c2, 0, :]
            return jnp.where(lr0 > 0.0, w, 0.0), w

        def column_iou(j, accs):
            noobj_a, loc_a, pobj_a, iou_a = accs
            r = lax.div(j, s2)
            c2 = lax.rem(j, s2)
            m, w = col_mask(r, c2)
            obj = m > 0.0
            lr0 = lr_v[r, c2, 0, :]
            lr1 = lr_v[r, c2, 1, :]
            pr0 = pr_v[r, c2, 0, :]
            pr1 = pr_v[r, c2, 1, :]

            def corners(ref, k0):
                x = ref[r, c2, k0, :]
                y = ref[r, c2, k0 + 1, :]
                bw = ref[r, c2, k0 + 2, :]
                bh = ref[r, c2, k0 + 3, :]
                hw = 0.5 * (bw * bw)
                hh = 0.5 * (bh * bh)
                return x - hw, y - hh, x + hw, y + hh

            def iou_for(k0):
                tx1, ty1, tx2, ty2 = corners(lb_v, k0)
                px1, py1, px2, py2 = corners(pb_v, k0)
                ltx = jnp.maximum(tx1, px1)
                lty = jnp.maximum(ty1, py1)
                rbx = jnp.minimum(tx2, px2)
                rby = jnp.minimum(ty2, py2)
                wx = jnp.maximum(rbx - ltx, 0.0)
                wy = jnp.maximum(rby - lty, 0.0)
                inter = wx * wy
                a1 = (tx2 - tx1) * (ty2 - ty1)
                a2 = (px2 - px1) * (py2 - py1)
                return jnp.where(obj, inter / (a1 + a2 - inter), 0.0)

            iou0 = iou_for(0)
            iou1 = iou_for(4)
            maxiou = jnp.maximum(iou0, iou1)
            sel1 = iou1 > iou0

            def sel(a0, a1):
                return jnp.where(sel1, a1, a0)

            loc_t = zero
            for k in range(4):
                dk = (sel(pb_v[r, c2, k, :], pb_v[r, c2, 4 + k, :])
                      - sel(lb_v[r, c2, k, :], lb_v[r, c2, 4 + k, :]))
                loc_t = loc_t + dk * dk
            loc_a = loc_a + m * loc_t
            dpo = sel(pr0, pr1) - maxiou
            dio = maxiou - sel(lr0, lr1)
            pobj_a = pobj_a + m * (dpo * dpo)
            iou_a = iou_a + m * (dio * dio)
            nm = w - m
            d0 = pr0 - lr0
            d1 = pr1 - lr1
            noobj_a = noobj_a + nm * (d0 * d0 + d1 * d1)
            return noobj_a, loc_a, pobj_a, iou_a

        accs = lax.fori_loop(0, ncols, column_iou, (zero, zero, zero, zero))
        noobj_a, loc_a, pobj_a, iou_a = accs

        c_pc.wait()
        c_lc.wait()

        def column_cls(j, cls_a):
            r = lax.div(j, s2)
            c2 = lax.rem(j, s2)
            m, _ = col_mask(r, c2)
            csum = zero
            for c in range(CLS):
                d = pc_v[r, c, c2, :] - lc_v[r, c, c2, :]
                csum = csum + d * d
            return cls_a + m * csum

        cls_a = lax.fori_loop(0, ncols, column_cls, zero)

        inv = 1.0 / batch
        s_off = (L_COORD * inv) * jnp.sum(loc_a)
        s_cls = inv * jnp.sum(cls_a)
        s_pobj = inv * jnp.sum(pobj_a)
        s_nobj = (L_NOOBJ * inv) * jnp.sum(noobj_a)
        s_iou = inv * jnp.sum(iou_a)
        res = (jnp.where(lanes == 0, s_off, 0.0)
               + jnp.where(lanes == 1, s_cls, 0.0)
               + jnp.where(lanes == 2, s_pobj, 0.0)
               + jnp.where(lanes == 3, s_nobj, 0.0)
               + jnp.where(lanes == 4, s_iou, 0.0))
        part_v[...] = res
        pltpu.sync_copy(part_v, out_hbm.at[wid])

    return pl.kernel(
        body,
        out_type=jax.ShapeDtypeStruct((NC * NS, L), jnp.float32),
        mesh=mesh,
        scratch_types=[
            pltpu.VMEM((rows, CLS, s2, L), jnp.float32),
            pltpu.VMEM((rows, CLS, s2, L), jnp.float32),
            pltpu.VMEM((rows, s2, 2, L), jnp.float32),
            pltpu.VMEM((rows, s2, 2, L), jnp.float32),
            pltpu.VMEM((rows, s2, 8, L), jnp.float32),
            pltpu.VMEM((rows, s2, 8, L), jnp.float32),
            pltpu.VMEM((L,), jnp.float32),
            pltpu.SemaphoreType.DMA,
            pltpu.SemaphoreType.DMA,
        ],
        compiler_params=pltpu.CompilerParams(needs_layout_passes=False,
                                             use_tc_tiling_on_sc=False),
    )


def _resp_view(x):
    k = x.shape[-1]
    return jnp.transpose(x.reshape(2, 128, 7, 7, k), (2, 3, 0, 4, 1))


def kernel(pred_cls, pred_response, pred_bboxes, label_cls, label_response,
           label_bboxes):
    batch, s1, s2 = pred_cls.shape[0], pred_cls.shape[1], pred_cls.shape[2]
    fn = _build_sc_loss(batch, s1, s2)
    cls_st = jnp.stack([jnp.transpose(pred_cls, (1, 3, 2, 0)),
                        jnp.transpose(label_cls, (1, 3, 2, 0))])
    resp_st = jnp.stack([_resp_view(pred_response),
                         _resp_view(label_response)])
    out = fn(cls_st, resp_st, _resp_view(pred_bboxes),
             _resp_view(label_bboxes))
    s = jnp.sum(out, axis=0)
    return {"offset": s[0], "cls": s[1], "pObj": s[2], "nObj": s[3],
            "iou": s[4]}
